# Initial kernel scaffold; baseline (speedup 1.0000x reference)
import jax
import jax.numpy as jnp
from jax.experimental import pallas as pl


def kernel(P_batch, Q_batch, thetas):
    raise NotImplementedError("write your pallas kernel here")


# speedup vs baseline: 120.7510x; 120.7510x over previous
import functools

import jax
import jax.numpy as jnp
from jax import lax
from jax.experimental import pallas as pl
from jax.experimental.pallas import tpu as pltpu
from jax.experimental.pallas import tpu_sc as plsc

_BS = 16
_N = 8192
_L = 100
_LANES = 16
_VREGS = _N // _LANES
_LEVELS = 9


def _sort_body(pq_hbm, out_hbm, buf):
    wid = lax.axis_index("s") * 2 + lax.axis_index("c")
    pltpu.sync_copy(pq_hbm.at[wid], buf)

    def vsort_pass(lvl):
        if lvl >= _LEVELS:
            def body(i, carry):
                v = buf[pl.ds(i * _LANES, _LANES)]
                sk, _ = plsc.sort_key_val(v, v)
                buf[pl.ds(i * _LANES, _LANES)] = sk
                return carry

            lax.fori_loop(0, _VREGS, body, 0)
        else:
            half = 1 << lvl

            def body(m, carry, lvl=lvl, half=half):
                grp = m >> lvl
                off = m & (half - 1)
                ia = (grp << (lvl + 1)) + off
                idd = ia + half
                va = buf[pl.ds(ia * _LANES, _LANES)]
                vd = buf[pl.ds(idd * _LANES, _LANES)]
                sa, _ = plsc.sort_key_val(va, va)
                sd, _ = plsc.sort_key_val(vd, vd, descending=True)
                buf[pl.ds(ia * _LANES, _LANES)] = sa
                buf[pl.ds(idd * _LANES, _LANES)] = sd
                return carry

            lax.fori_loop(0, _VREGS // 2, body, 0)

    vsort_pass(0)
    for lvl in range(1, _LEVELS + 1):
        for tlog in range(lvl - 1, -1, -1):
            t = 1 << tlog

            def body(p, carry, tlog=tlog, t=t, lvl=lvl):
                q = p >> tlog
                r = p & (t - 1)
                i = (q << (tlog + 1)) + r
                j = i + t
                asc = ((i >> lvl) & 1) == 0
                a = buf[pl.ds(i * _LANES, _LANES)]
                b = buf[pl.ds(j * _LANES, _LANES)]
                lo = jnp.minimum(a, b)
                hi = jnp.maximum(a, b)
                ilo = jnp.where(asc, i, j)
                ihi = jnp.where(asc, j, i)
                buf[pl.ds(ilo * _LANES, _LANES)] = lo
                buf[pl.ds(ihi * _LANES, _LANES)] = hi
                return carry

            lax.fori_loop(0, _VREGS // 2, body, 0)
        vsort_pass(lvl)

    pltpu.sync_copy(buf, out_hbm.at[wid])


_sort_call = functools.partial(
    pl.kernel,
    out_type=jax.ShapeDtypeStruct((2 * _BS, _N), jnp.float32),
    mesh=plsc.VectorSubcoreMesh(core_axis_name="c", subcore_axis_name="s"),
    scratch_types=[pltpu.VMEM((_N,), jnp.float32)],
    compiler_params=pltpu.CompilerParams(needs_layout_passes=False),
)(_sort_body)


def _reduce_body(sp_ref, sq_ref, th_ref, out_ref):
    d = sp_ref[...] - sq_ref[...]
    d2 = jnp.sum(d * d, axis=1) * jnp.float32(1.0 / _N)
    t = th_ref[...]
    tn = t / jnp.sqrt(t * t)
    f = jnp.sum(tn * tn, axis=1) * jnp.float32(1.0 / _L)
    out_ref[...] = jnp.sum(jnp.sqrt(d2 * f)).reshape(1, 1)


def kernel(P_batch, Q_batch, thetas):
    pq = jnp.concatenate([P_batch, Q_batch], axis=0)
    sorted_pq = _sort_call(pq)
    sp = sorted_pq[:_BS]
    sq = sorted_pq[_BS:]
    th = thetas.reshape(_BS, _L)
    out = pl.pallas_call(
        _reduce_body,
        out_shape=jax.ShapeDtypeStruct((1, 1), jnp.float32),
    )(sp, sq, th)
    return out[0, 0]


# speedup vs baseline: 270.9855x; 2.2442x over previous
import functools

import jax
import jax.numpy as jnp
from jax import lax
from jax.experimental import pallas as pl
from jax.experimental.pallas import tpu as pltpu
from jax.experimental.pallas import tpu_sc as plsc

_BS = 16
_N = 8192
_L = 100
_LANES = 16
_VREGS = _N // _LANES
_LEVELS = 9


def _sort_body(pq_hbm, out_hbm, buf):
    wid = lax.axis_index("s") * 2 + lax.axis_index("c")
    pltpu.sync_copy(pq_hbm.at[wid], buf)

    def vsort_pass(lvl):
        if lvl >= _LEVELS:
            @plsc.parallel_loop(0, _VREGS, unroll=8)
            def _(i):
                v = buf[pl.ds(i * _LANES, _LANES)]
                sk, _ = plsc.sort_key_val(v, v)
                buf[pl.ds(i * _LANES, _LANES)] = sk

        else:
            half = 1 << lvl

            @plsc.parallel_loop(0, _VREGS // 2, unroll=4)
            def _(m, lvl=lvl, half=half):
                grp = m >> lvl
                off = m & (half - 1)
                ia = (grp << (lvl + 1)) + off
                idd = ia + half
                va = buf[pl.ds(ia * _LANES, _LANES)]
                vd = buf[pl.ds(idd * _LANES, _LANES)]
                sa, _ = plsc.sort_key_val(va, va)
                sd, _ = plsc.sort_key_val(vd, vd, descending=True)
                buf[pl.ds(ia * _LANES, _LANES)] = sa
                buf[pl.ds(idd * _LANES, _LANES)] = sd

    vsort_pass(0)
    for lvl in range(1, _LEVELS + 1):
        for tlog in range(lvl - 1, -1, -1):
            t = 1 << tlog

            @plsc.parallel_loop(0, _VREGS // 2, unroll=4)
            def _(p, tlog=tlog, t=t, lvl=lvl):
                q = p >> tlog
                r = p & (t - 1)
                i = (q << (tlog + 1)) + r
                j = i + t
                asc = ((i >> lvl) & 1) == 0
                a = buf[pl.ds(i * _LANES, _LANES)]
                b = buf[pl.ds(j * _LANES, _LANES)]
                lo = jnp.minimum(a, b)
                hi = jnp.maximum(a, b)
                ilo = jnp.where(asc, i, j)
                ihi = jnp.where(asc, j, i)
                buf[pl.ds(ilo * _LANES, _LANES)] = lo
                buf[pl.ds(ihi * _LANES, _LANES)] = hi
        vsort_pass(lvl)

    pltpu.sync_copy(buf, out_hbm.at[wid])


_sort_call = functools.partial(
    pl.kernel,
    out_type=jax.ShapeDtypeStruct((2 * _BS, _N), jnp.float32),
    mesh=plsc.VectorSubcoreMesh(core_axis_name="c", subcore_axis_name="s"),
    scratch_types=[pltpu.VMEM((_N,), jnp.float32)],
    compiler_params=pltpu.CompilerParams(needs_layout_passes=False),
)(_sort_body)


def _reduce_body(sp_ref, sq_ref, th_ref, out_ref):
    d = sp_ref[...] - sq_ref[...]
    d2 = jnp.sum(d * d, axis=1) * jnp.float32(1.0 / _N)
    t = th_ref[...]
    tn = t / jnp.sqrt(t * t)
    f = jnp.sum(tn * tn, axis=1) * jnp.float32(1.0 / _L)
    out_ref[...] = jnp.sum(jnp.sqrt(d2 * f)).reshape(1, 1)


def kernel(P_batch, Q_batch, thetas):
    pq = jnp.concatenate([P_batch, Q_batch], axis=0)
    sorted_pq = _sort_call(pq)
    sp = sorted_pq[:_BS]
    sq = sorted_pq[_BS:]
    th = thetas.reshape(_BS, _L)
    out = pl.pallas_call(
        _reduce_body,
        out_shape=jax.ShapeDtypeStruct((1, 1), jnp.float32),
    )(sp, sq, th)
    return out[0, 0]


# speedup vs baseline: 337.2797x; 1.2446x over previous
import functools

import jax
import jax.numpy as jnp
from jax import lax
from jax.experimental import pallas as pl
from jax.experimental.pallas import tpu as pltpu
from jax.experimental.pallas import tpu_sc as plsc

_BS = 16
_N = 8192
_L = 100
_LANES = 16
_VREGS = _N // _LANES
_LEVELS = 9


def _sort_body(pq_hbm, out_hbm, buf):
    wid = lax.axis_index("s") * 2 + lax.axis_index("c")
    pltpu.sync_copy(pq_hbm.at[wid], buf)

    def _ld(i):
        return buf[pl.ds(i * _LANES, _LANES)]

    def _st(i, v):
        buf[pl.ds(i * _LANES, _LANES)] = v

    @plsc.parallel_loop(0, _VREGS // 2, unroll=4)
    def _(m):
        i = m << 1
        a = _ld(i)
        b = _ld(i + 1)
        sa, _ = plsc.sort_key_val(a, a)
        sd, _ = plsc.sort_key_val(b, b, descending=True)
        _st(i, sa)
        _st(i + 1, sd)

    for lvl in range(1, _LEVELS + 1):
        final = lvl >= _LEVELS
        for tlog in range(lvl - 1, 0, -1):
            t = 1 << tlog
            if final:

                @plsc.parallel_loop(0, _VREGS // 2, unroll=4)
                def _(p, tlog=tlog, t=t):
                    q = p >> tlog
                    r = p & (t - 1)
                    i = (q << (tlog + 1)) + r
                    a = _ld(i)
                    b = _ld(i + t)
                    _st(i, jnp.minimum(a, b))
                    _st(i + t, jnp.maximum(a, b))

            else:

                @plsc.parallel_loop(0, _VREGS // 4, unroll=4)
                def _(m, tlog=tlog, t=t, lvl=lvl):
                    rp = m >> (lvl - 1)
                    w = m & ((1 << (lvl - 1)) - 1)
                    q = w >> tlog
                    r = w & (t - 1)
                    ia = (rp << (lvl + 1)) + (q << (tlog + 1)) + r
                    idd = ia + (1 << lvl)
                    a = _ld(ia)
                    b = _ld(ia + t)
                    _st(ia, jnp.minimum(a, b))
                    _st(ia + t, jnp.maximum(a, b))
                    c = _ld(idd)
                    d = _ld(idd + t)
                    _st(idd, jnp.maximum(c, d))
                    _st(idd + t, jnp.minimum(c, d))

        if final:

            @plsc.parallel_loop(0, _VREGS // 2, unroll=4)
            def _(p):
                i = p << 1
                a = _ld(i)
                b = _ld(i + 1)
                slo, _ = plsc.sort_key_val(jnp.minimum(a, b), a)
                shi, _ = plsc.sort_key_val(jnp.maximum(a, b), a)
                _st(i, slo)
                _st(i + 1, shi)

        else:

            @plsc.parallel_loop(0, _VREGS // 4, unroll=2)
            def _(m, lvl=lvl):
                rp = m >> (lvl - 1)
                w = m & ((1 << (lvl - 1)) - 1)
                ia = (rp << (lvl + 1)) + (w << 1)
                idd = ia + (1 << lvl)
                a = _ld(ia)
                b = _ld(ia + 1)
                slo, _ = plsc.sort_key_val(jnp.minimum(a, b), a)
                shi, _ = plsc.sort_key_val(jnp.maximum(a, b), a)
                _st(ia, slo)
                _st(ia + 1, shi)
                c = _ld(idd)
                d = _ld(idd + 1)
                shi2, _ = plsc.sort_key_val(jnp.maximum(c, d), c, descending=True)
                slo2, _ = plsc.sort_key_val(jnp.minimum(c, d), c, descending=True)
                _st(idd, shi2)
                _st(idd + 1, slo2)

    pltpu.sync_copy(buf, out_hbm.at[wid])


_sort_call = functools.partial(
    pl.kernel,
    out_type=jax.ShapeDtypeStruct((2 * _BS, _N), jnp.float32),
    mesh=plsc.VectorSubcoreMesh(core_axis_name="c", subcore_axis_name="s"),
    scratch_types=[pltpu.VMEM((_N,), jnp.float32)],
    compiler_params=pltpu.CompilerParams(needs_layout_passes=False),
)(_sort_body)


def _reduce_body(sp_ref, sq_ref, th_ref, out_ref):
    d = sp_ref[...] - sq_ref[...]
    d2 = jnp.sum(d * d, axis=1) * jnp.float32(1.0 / _N)
    t = th_ref[...]
    tn = t / jnp.sqrt(t * t)
    f = jnp.sum(tn * tn, axis=1) * jnp.float32(1.0 / _L)
    out_ref[...] = jnp.sum(jnp.sqrt(d2 * f)).reshape(1, 1)


def kernel(P_batch, Q_batch, thetas):
    pq = jnp.concatenate([P_batch, Q_batch], axis=0)
    sorted_pq = _sort_call(pq)
    sp = sorted_pq[:_BS]
    sq = sorted_pq[_BS:]
    th = thetas.reshape(_BS, _L)
    out = pl.pallas_call(
        _reduce_body,
        out_shape=jax.ShapeDtypeStruct((1, 1), jnp.float32),
    )(sp, sq, th)
    return out[0, 0]


# speedup vs baseline: 355.0787x; 1.0528x over previous
import functools

import jax
import jax.numpy as jnp
from jax import lax
from jax.experimental import pallas as pl
from jax.experimental.pallas import tpu as pltpu
from jax.experimental.pallas import tpu_sc as plsc

_BS = 16
_N = 8192
_L = 100
_LANES = 16
_VREGS = _N // _LANES
_LEVELS = 9


def _sort_body(p_hbm, q_hbm, out_hbm, buf):
    wid = lax.axis_index("s") * 2 + lax.axis_index("c")

    @pl.when(wid < _BS)
    def _():
        pltpu.sync_copy(p_hbm.at[wid], buf)

    @pl.when(wid >= _BS)
    def _():
        pltpu.sync_copy(q_hbm.at[wid - _BS], buf)

    def _ld(i):
        return buf[pl.ds(i * _LANES, _LANES)]

    def _st(i, v):
        buf[pl.ds(i * _LANES, _LANES)] = v

    @plsc.parallel_loop(0, _VREGS // 2, unroll=4)
    def _(m):
        i = m << 1
        a = _ld(i)
        b = _ld(i + 1)
        sa, _ = plsc.sort_key_val(a, a)
        sd, _ = plsc.sort_key_val(b, b, descending=True)
        _st(i, sa)
        _st(i + 1, sd)

    for lvl in range(1, _LEVELS + 1):
        final = lvl >= _LEVELS
        for tlog in range(lvl - 1, 0, -1):
            t = 1 << tlog
            if final:

                @plsc.parallel_loop(0, _VREGS // 2, unroll=4)
                def _(p, tlog=tlog, t=t):
                    q = p >> tlog
                    r = p & (t - 1)
                    i = (q << (tlog + 1)) + r
                    a = _ld(i)
                    b = _ld(i + t)
                    _st(i, jnp.minimum(a, b))
                    _st(i + t, jnp.maximum(a, b))

            else:

                @plsc.parallel_loop(0, _VREGS // 4, unroll=4)
                def _(m, tlog=tlog, t=t, lvl=lvl):
                    rp = m >> (lvl - 1)
                    w = m & ((1 << (lvl - 1)) - 1)
                    q = w >> tlog
                    r = w & (t - 1)
                    ia = (rp << (lvl + 1)) + (q << (tlog + 1)) + r
                    idd = ia + (1 << lvl)
                    a = _ld(ia)
                    b = _ld(ia + t)
                    _st(ia, jnp.minimum(a, b))
                    _st(ia + t, jnp.maximum(a, b))
                    c = _ld(idd)
                    d = _ld(idd + t)
                    _st(idd, jnp.maximum(c, d))
                    _st(idd + t, jnp.minimum(c, d))

        if final:

            @plsc.parallel_loop(0, _VREGS // 2, unroll=4)
            def _(p):
                i = p << 1
                a = _ld(i)
                b = _ld(i + 1)
                slo, _ = plsc.sort_key_val(jnp.minimum(a, b), a)
                shi, _ = plsc.sort_key_val(jnp.maximum(a, b), a)
                _st(i, slo)
                _st(i + 1, shi)

        else:

            @plsc.parallel_loop(0, _VREGS // 4, unroll=2)
            def _(m, lvl=lvl):
                rp = m >> (lvl - 1)
                w = m & ((1 << (lvl - 1)) - 1)
                ia = (rp << (lvl + 1)) + (w << 1)
                idd = ia + (1 << lvl)
                a = _ld(ia)
                b = _ld(ia + 1)
                slo, _ = plsc.sort_key_val(jnp.minimum(a, b), a)
                shi, _ = plsc.sort_key_val(jnp.maximum(a, b), a)
                _st(ia, slo)
                _st(ia + 1, shi)
                c = _ld(idd)
                d = _ld(idd + 1)
                shi2, _ = plsc.sort_key_val(jnp.maximum(c, d), c, descending=True)
                slo2, _ = plsc.sort_key_val(jnp.minimum(c, d), c, descending=True)
                _st(idd, shi2)
                _st(idd + 1, slo2)

    pltpu.sync_copy(buf, out_hbm.at[wid])


_sort_call = functools.partial(
    pl.kernel,
    out_type=jax.ShapeDtypeStruct((2 * _BS, _N), jnp.float32),
    mesh=plsc.VectorSubcoreMesh(core_axis_name="c", subcore_axis_name="s"),
    scratch_types=[pltpu.VMEM((_N,), jnp.float32)],
    compiler_params=pltpu.CompilerParams(needs_layout_passes=False),
)(_sort_body)


def _reduce_body(spq_ref, th_ref, out_ref):
    d = spq_ref[0:_BS, :] - spq_ref[_BS : 2 * _BS, :]
    d2 = jnp.sum(d * d, axis=1) * jnp.float32(1.0 / _N)
    t = th_ref[...]
    tn = t / jnp.sqrt(t * t)
    f = jnp.sum(tn * tn, axis=1) * jnp.float32(1.0 / _L)
    out_ref[...] = jnp.sum(jnp.sqrt(d2 * f)).reshape(1, 1)


def kernel(P_batch, Q_batch, thetas):
    sorted_pq = _sort_call(P_batch, Q_batch)
    th = thetas.reshape(_BS, _L)
    out = pl.pallas_call(
        _reduce_body,
        out_shape=jax.ShapeDtypeStruct((1, 1), jnp.float32),
    )(sorted_pq, th)
    return out[0, 0]
